# baseline (speedup 1.0000x reference)
import dataclasses
import functools

import jax
import jax.numpy as jnp
from jax import lax
from jax.experimental import pallas as pl
from jax.experimental.pallas import tpu as pltpu
from jax.experimental.pallas import tpu_sc as plsc

_N = 100000
_NP = 100096
_E = 6400000
_CHUNK = 2048
_ROWS = _CHUNK // 128
_NCHUNKS = _E // _CHUNK
_NW = 32
_NPT = _NP // 16
_NSLOT = -(-_NCHUNKS // _NW)


def _sc_forces(xp, yp, zp, ex, ey, ez, zeros1, a2, b2, f1):
    mesh = plsc.VectorSubcoreMesh(core_axis_name="c", subcore_axis_name="s")
    cp = pltpu.CompilerParams()
    if "needs_layout_passes" in pltpu.CompilerParams.__dataclass_fields__:
        cp = dataclasses.replace(cp, needs_layout_passes=False,
                                 use_tc_tiling_on_sc=False)

    f32 = jnp.float32
    scratch = []
    for _ in range(3):
        scratch.append(pltpu.VMEM((_ROWS, 128), jnp.int32))
        scratch.append(pltpu.VMEM((_ROWS, 128), jnp.int32))
        for _ in range(13):
            scratch.append(pltpu.VMEM((_CHUNK,), f32))
    for _ in range(6):
        scratch.append(pltpu.VMEM_SHARED((_NP,), f32))
    for _ in range(9):
        scratch.append(pltpu.SemaphoreType.DMA)

    @functools.partial(
        pl.kernel,
        mesh=mesh,
        compiler_params=cp,
        out_type=jax.ShapeDtypeStruct((6 * _NP,), f32),
        scratch_types=scratch,
    )
    def k(x_hbm, y_hbm, z_hbm, ex_hbm, ey_hbm, ez_hbm, zer_hbm,
          a_hbm, b_hbm, f_hbm, out_hbm, *scr):
        sets = [scr[i * 15:(i + 1) * 15] for i in range(3)]
        psx, psy, psz, asx, asy, asz = scr[45:51]
        sems = scr[51:60]
        SLD = sems[0:3]
        SG = sems[3:6]
        SS = sems[6:9]

        c = lax.axis_index("c")
        s = lax.axis_index("s")
        wid = c * 16 + s
        r0 = s * _NPT
        sl = pl.ds(r0, _NPT)

        pltpu.sync_copy(x_hbm.at[sl], psx.at[sl])
        pltpu.sync_copy(y_hbm.at[sl], psy.at[sl])
        pltpu.sync_copy(z_hbm.at[sl], psz.at[sl])

        @pl.when(c == 0)
        def _():
            pltpu.sync_copy(ex_hbm.at[sl], asx.at[sl])
            pltpu.sync_copy(ey_hbm.at[sl], asy.at[sl])
            pltpu.sync_copy(ez_hbm.at[sl], asz.at[sl])

        @pl.when(c != 0)
        def _():
            pltpu.sync_copy(zer_hbm.at[sl], asx.at[sl])
            pltpu.sync_copy(zer_hbm.at[sl], asy.at[sl])
            pltpu.sync_copy(zer_hbm.at[sl], asz.at[sl])

        plsc.subcore_barrier()

        def cid(g):
            return g * _NW + wid

        def valid(g):
            return cid(g) < _NCHUNKS

        def lin_descs(g, S):
            aix, bix, fbuf = sets[S][0], sets[S][1], sets[S][2]
            ci = cid(g)
            return [
                pltpu.make_async_copy(
                    a_hbm.at[pl.ds(ci * _ROWS, _ROWS)], aix, SLD[S]),
                pltpu.make_async_copy(
                    b_hbm.at[pl.ds(ci * _ROWS, _ROWS)], bix, SLD[S]),
                pltpu.make_async_copy(
                    f_hbm.at[pl.ds(ci * _CHUNK, _CHUNK)], fbuf, SLD[S]),
            ]

        def gat_descs(S):
            aix, bix = sets[S][0], sets[S][1]
            pxa, pya, pza, pxb, pyb, pzb = sets[S][3:9]
            out = []
            for j in range(_ROWS):
                dst = pl.ds(j * 128, 128)
                ia = aix.at[j]
                ib = bix.at[j]
                out.append(pltpu.make_async_copy(psx.at[ia], pxa.at[dst], SG[S]))
                out.append(pltpu.make_async_copy(psy.at[ia], pya.at[dst], SG[S]))
                out.append(pltpu.make_async_copy(psz.at[ia], pza.at[dst], SG[S]))
                out.append(pltpu.make_async_copy(psx.at[ib], pxb.at[dst], SG[S]))
                out.append(pltpu.make_async_copy(psy.at[ib], pyb.at[dst], SG[S]))
                out.append(pltpu.make_async_copy(psz.at[ib], pzb.at[dst], SG[S]))
            return out

        def sca_descs(S):
            aix, bix = sets[S][0], sets[S][1]
            fax, fay, faz, fbx, fby, fbz = sets[S][9:15]
            out = []
            for j in range(_ROWS):
                src = pl.ds(j * 128, 128)
                ia = aix.at[j]
                ib = bix.at[j]
                out.append(pltpu.make_async_copy(fax.at[src], asx.at[ia], SS[S]))
                out.append(pltpu.make_async_copy(fay.at[src], asy.at[ia], SS[S]))
                out.append(pltpu.make_async_copy(faz.at[src], asz.at[ia], SS[S]))
                out.append(pltpu.make_async_copy(fbx.at[src], asx.at[ib], SS[S]))
                out.append(pltpu.make_async_copy(fby.at[src], asy.at[ib], SS[S]))
                out.append(pltpu.make_async_copy(fbz.at[src], asz.at[ib], SS[S]))
            return out

        def fire(descs, add=False):
            for d in descs:
                d.start(add=add)

        def drain(descs):
            for d in descs:
                d.wait()

        def compute(S):
            fbuf = sets[S][2]
            pxa, pya, pza, pxb, pyb, pzb = sets[S][3:9]
            fax, fay, faz, fbx, fby, fbz = sets[S][9:15]

            @pl.loop(0, _CHUNK // 16)
            def _(r):
                q = pl.ds(r * 16, 16)
                vx = pxb[q] - pxa[q]
                vy = pyb[q] - pya[q]
                vz = pzb[q] - pza[q]
                d = vx * vx + vy * vy + vz * vz
                bits = lax.bitcast_convert_type(d, jnp.int32)
                y = lax.bitcast_convert_type(
                    jnp.int32(0x5F3759DF) - (bits >> 1), f32)
                y = y * (1.5 - 0.5 * d * y * y)
                y = y * (1.5 - 0.5 * d * y * y)
                y = y * (1.5 - 0.5 * d * y * y)
                sp = fbuf[q] * y
                gx = sp * vx
                gy = sp * vy
                gz = sp * vz
                fbx[q] = gx
                fby[q] = gy
                fbz[q] = gz
                fax[q] = -gx
                fay[q] = -gy
                faz[q] = -gz

        def body(g, S):
            Snext = (S + 1) % 3

            @pl.when((g >= 2) & valid(g - 2))
            def _():
                drain(sca_descs(Snext))

            @pl.when(valid(g + 1))
            def _():
                fire(lin_descs(g + 1, Snext))

            @pl.when(valid(g))
            def _():
                drain(gat_descs(S))

            compute(S)

            @pl.when(valid(g))
            def _():
                fire(sca_descs(S), add=True)

            @pl.when(valid(g + 1))
            def _():
                drain(lin_descs(g + 1, Snext))
                fire(gat_descs(Snext))

        fire(lin_descs(0, 0))
        drain(lin_descs(0, 0))
        fire(gat_descs(0))

        @pl.loop(0, (_NSLOT - 2) // 3)
        def _(t):
            g = t * 3
            body(g, 0)
            body(g + 1, 1)
            body(g + 2, 2)

        body(_NSLOT - 2, 0)
        body(_NSLOT - 1, 1)

        @pl.when(valid(_NSLOT - 2))
        def _():
            drain(sca_descs(0))

        @pl.when(valid(_NSLOT - 1))
        def _():
            drain(sca_descs(1))

        plsc.subcore_barrier()
        base = c * 3 * _NP
        pltpu.sync_copy(asx.at[sl], out_hbm.at[pl.ds(base + r0, _NPT)])
        pltpu.sync_copy(asy.at[sl], out_hbm.at[pl.ds(base + _NP + r0, _NPT)])
        pltpu.sync_copy(asz.at[sl],
                        out_hbm.at[pl.ds(base + 2 * _NP + r0, _NPT)])

    return k(xp, yp, zp, ex, ey, ez, zeros1, a2, b2, f1)


def _tc_combine(p0, p1):
    def body(x_ref, y_ref, o_ref):
        o_ref[...] = x_ref[...] + y_ref[...]

    return pl.pallas_call(
        body,
        out_shape=jax.ShapeDtypeStruct(p0.shape, p0.dtype),
    )(p0, p1)


def kernel(points, external_forces, force, edge_index):
    pad = (0, _NP - _N)
    xp = jnp.pad(points[:, 0], pad)
    yp = jnp.pad(points[:, 1], pad)
    zp = jnp.pad(points[:, 2], pad)
    ex = jnp.pad(external_forces[:, 0], pad)
    ey = jnp.pad(external_forces[:, 1], pad)
    ez = jnp.pad(external_forces[:, 2], pad)
    zeros1 = jnp.zeros((_NP,), jnp.float32)
    a2 = edge_index[0].reshape(_E // 128, 128)
    b2 = edge_index[1].reshape(_E // 128, 128)
    partial = _sc_forces(xp, yp, zp, ex, ey, ez, zeros1, a2, b2, force)
    m = 3 * _NP // 128
    s = _tc_combine(partial[:3 * _NP].reshape(m, 128),
                    partial[3 * _NP:].reshape(m, 128))
    return s.reshape(3, _NP)[:, :_N].T


# speedup vs baseline: 1.0899x; 1.0899x over previous
import dataclasses
import functools

import jax
import jax.numpy as jnp
from jax import lax
from jax.experimental import pallas as pl
from jax.experimental.pallas import tpu as pltpu
from jax.experimental.pallas import tpu_sc as plsc

_N = 100000
_NP = 100096
_E = 6400000
_CHUNK = 2048
_ROWS = _CHUNK // 128
_NCHUNKS = _E // _CHUNK
_NW = 32
_NPT = _NP // 16


def _sc_forces(xp, yp, zp, ex, ey, ez, zeros1, a2, b2, f1):
    mesh = plsc.VectorSubcoreMesh(core_axis_name="c", subcore_axis_name="s")
    cp = pltpu.CompilerParams()
    if "needs_layout_passes" in pltpu.CompilerParams.__dataclass_fields__:
        cp = dataclasses.replace(cp, needs_layout_passes=False,
                                 use_tc_tiling_on_sc=False)

    f32 = jnp.float32
    scratch = [pltpu.VMEM((_ROWS, 128), jnp.int32),
               pltpu.VMEM((_ROWS, 128), jnp.int32)]
    for _ in range(13):
        scratch.append(pltpu.VMEM((_CHUNK,), f32))
    for _ in range(6):
        scratch.append(pltpu.VMEM_SHARED((_NP,), f32))
    scratch.append(pltpu.SemaphoreType.DMA)
    scratch.append(pltpu.SemaphoreType.DMA)

    @functools.partial(
        pl.kernel,
        mesh=mesh,
        compiler_params=cp,
        out_type=jax.ShapeDtypeStruct((6 * _NP,), f32),
        scratch_types=scratch,
    )
    def k(x_hbm, y_hbm, z_hbm, ex_hbm, ey_hbm, ez_hbm, zer_hbm,
          a_hbm, b_hbm, f_hbm, out_hbm, *scr):
        aix, bix, fbuf = scr[0:3]
        pxa, pya, pza, pxb, pyb, pzb = scr[3:9]
        fax, fay, faz, fbx, fby, fbz = scr[9:15]
        psx, psy, psz, asx, asy, asz = scr[15:21]
        sem_g, sem_s = scr[21:23]

        c = lax.axis_index("c")
        s = lax.axis_index("s")
        wid = c * 16 + s
        r0 = s * _NPT
        sl = pl.ds(r0, _NPT)

        pltpu.sync_copy(x_hbm.at[sl], psx.at[sl])
        pltpu.sync_copy(y_hbm.at[sl], psy.at[sl])
        pltpu.sync_copy(z_hbm.at[sl], psz.at[sl])

        @pl.when(c == 0)
        def _():
            pltpu.sync_copy(ex_hbm.at[sl], asx.at[sl])
            pltpu.sync_copy(ey_hbm.at[sl], asy.at[sl])
            pltpu.sync_copy(ez_hbm.at[sl], asz.at[sl])

        @pl.when(c != 0)
        def _():
            pltpu.sync_copy(zer_hbm.at[sl], asx.at[sl])
            pltpu.sync_copy(zer_hbm.at[sl], asy.at[sl])
            pltpu.sync_copy(zer_hbm.at[sl], asz.at[sl])

        plsc.subcore_barrier()

        rem = _NCHUNKS % _NW
        ng = jnp.where(wid < rem, _NCHUNKS // _NW + 1, _NCHUNKS // _NW)

        @pl.loop(0, ng)
        def _(g):
            cid = g * _NW + wid
            pltpu.sync_copy(a_hbm.at[pl.ds(cid * _ROWS, _ROWS)], aix)
            pltpu.sync_copy(b_hbm.at[pl.ds(cid * _ROWS, _ROWS)], bix)
            pltpu.sync_copy(f_hbm.at[pl.ds(cid * _CHUNK, _CHUNK)], fbuf)

            gathers = []
            for j in range(_ROWS):
                dst = pl.ds(j * 128, 128)
                ia = aix.at[j]
                ib = bix.at[j]
                gathers.append(pltpu.async_copy(psx.at[ia], pxa.at[dst], sem_g))
                gathers.append(pltpu.async_copy(psy.at[ia], pya.at[dst], sem_g))
                gathers.append(pltpu.async_copy(psz.at[ia], pza.at[dst], sem_g))
                gathers.append(pltpu.async_copy(psx.at[ib], pxb.at[dst], sem_g))
                gathers.append(pltpu.async_copy(psy.at[ib], pyb.at[dst], sem_g))
                gathers.append(pltpu.async_copy(psz.at[ib], pzb.at[dst], sem_g))
            for h in gathers:
                h.wait()

            @pl.loop(0, _CHUNK // 16)
            def _(r):
                q = pl.ds(r * 16, 16)
                vx = pxb[q] - pxa[q]
                vy = pyb[q] - pya[q]
                vz = pzb[q] - pza[q]
                d = vx * vx + vy * vy + vz * vz
                bits = lax.bitcast_convert_type(d, jnp.int32)
                y = lax.bitcast_convert_type(
                    jnp.int32(0x5F3759DF) - (bits >> 1), f32)
                y = y * (1.5 - 0.5 * d * y * y)
                y = y * (1.5 - 0.5 * d * y * y)
                y = y * (1.5 - 0.5 * d * y * y)
                sp = fbuf[q] * y
                gx = sp * vx
                gy = sp * vy
                gz = sp * vz
                fbx[q] = gx
                fby[q] = gy
                fbz[q] = gz
                fax[q] = -gx
                fay[q] = -gy
                faz[q] = -gz

            scatters = []
            for j in range(_ROWS):
                src = pl.ds(j * 128, 128)
                ia = aix.at[j]
                ib = bix.at[j]
                scatters.append(pltpu.async_copy(
                    fax.at[src], asx.at[ia], sem_s, add=True))
                scatters.append(pltpu.async_copy(
                    fay.at[src], asy.at[ia], sem_s, add=True))
                scatters.append(pltpu.async_copy(
                    faz.at[src], asz.at[ia], sem_s, add=True))
                scatters.append(pltpu.async_copy(
                    fbx.at[src], asx.at[ib], sem_s, add=True))
                scatters.append(pltpu.async_copy(
                    fby.at[src], asy.at[ib], sem_s, add=True))
                scatters.append(pltpu.async_copy(
                    fbz.at[src], asz.at[ib], sem_s, add=True))
            for h in scatters:
                h.wait()

        plsc.subcore_barrier()
        base = c * 3 * _NP
        pltpu.sync_copy(asx.at[sl], out_hbm.at[pl.ds(base + r0, _NPT)])
        pltpu.sync_copy(asy.at[sl], out_hbm.at[pl.ds(base + _NP + r0, _NPT)])
        pltpu.sync_copy(asz.at[sl],
                        out_hbm.at[pl.ds(base + 2 * _NP + r0, _NPT)])

    return k(xp, yp, zp, ex, ey, ez, zeros1, a2, b2, f1)


def _tc_combine(p0, p1):
    def body(x_ref, y_ref, o_ref):
        o_ref[...] = x_ref[...] + y_ref[...]

    return pl.pallas_call(
        body,
        out_shape=jax.ShapeDtypeStruct(p0.shape, p0.dtype),
    )(p0, p1)


def kernel(points, external_forces, force, edge_index):
    pad = (0, _NP - _N)
    xp = jnp.pad(points[:, 0], pad)
    yp = jnp.pad(points[:, 1], pad)
    zp = jnp.pad(points[:, 2], pad)
    ex = jnp.pad(external_forces[:, 0], pad)
    ey = jnp.pad(external_forces[:, 1], pad)
    ez = jnp.pad(external_forces[:, 2], pad)
    zeros1 = jnp.zeros((_NP,), jnp.float32)
    a2 = edge_index[0].reshape(_E // 128, 128)
    b2 = edge_index[1].reshape(_E // 128, 128)
    partial = _sc_forces(xp, yp, zp, ex, ey, ez, zeros1, a2, b2, force)
    m = 3 * _NP // 128
    s = _tc_combine(partial[:3 * _NP].reshape(m, 128),
                    partial[3 * _NP:].reshape(m, 128))
    return s.reshape(3, _NP)[:, :_N].T


# speedup vs baseline: 1.1036x; 1.0125x over previous
import dataclasses
import functools

import jax
import jax.numpy as jnp
from jax import lax
from jax.experimental import pallas as pl
from jax.experimental.pallas import tpu as pltpu
from jax.experimental.pallas import tpu_sc as plsc

_N = 100000
_NP = 100096
_E = 6400000
_CHUNK = 2048
_ROWS = _CHUNK // 128
_NCHUNKS = _E // _CHUNK
_NW = 32
_NPT = _NP // 16


def _sc_forces(xp, yp, zp, ex, ey, ez, zeros1, a2, b2, f1):
    mesh = plsc.VectorSubcoreMesh(core_axis_name="c", subcore_axis_name="s")
    cp = pltpu.CompilerParams()
    if "needs_layout_passes" in pltpu.CompilerParams.__dataclass_fields__:
        cp = dataclasses.replace(cp, needs_layout_passes=False,
                                 use_tc_tiling_on_sc=False)

    f32 = jnp.float32
    scratch = [pltpu.VMEM((_CHUNK,), jnp.int32),
               pltpu.VMEM((_CHUNK,), jnp.int32)]
    for _ in range(13):
        scratch.append(pltpu.VMEM((_CHUNK,), f32))
    for _ in range(6):
        scratch.append(pltpu.VMEM_SHARED((_NP,), f32))
    scratch.append(pltpu.SemaphoreType.DMA)
    scratch.append(pltpu.SemaphoreType.DMA)

    @functools.partial(
        pl.kernel,
        mesh=mesh,
        compiler_params=cp,
        out_type=jax.ShapeDtypeStruct((6 * _NP,), f32),
        scratch_types=scratch,
    )
    def k(x_hbm, y_hbm, z_hbm, ex_hbm, ey_hbm, ez_hbm, zer_hbm,
          a_hbm, b_hbm, f_hbm, out_hbm, *scr):
        aix, bix, fbuf = scr[0:3]
        pxa, pya, pza, pxb, pyb, pzb = scr[3:9]
        fax, fay, faz, fbx, fby, fbz = scr[9:15]
        psx, psy, psz, asx, asy, asz = scr[15:21]
        sem_g, sem_s = scr[21:23]

        c = lax.axis_index("c")
        s = lax.axis_index("s")
        wid = c * 16 + s
        r0 = s * _NPT
        sl = pl.ds(r0, _NPT)

        pltpu.sync_copy(x_hbm.at[sl], psx.at[sl])
        pltpu.sync_copy(y_hbm.at[sl], psy.at[sl])
        pltpu.sync_copy(z_hbm.at[sl], psz.at[sl])

        @pl.when(c == 0)
        def _():
            pltpu.sync_copy(ex_hbm.at[sl], asx.at[sl])
            pltpu.sync_copy(ey_hbm.at[sl], asy.at[sl])
            pltpu.sync_copy(ez_hbm.at[sl], asz.at[sl])

        @pl.when(c != 0)
        def _():
            pltpu.sync_copy(zer_hbm.at[sl], asx.at[sl])
            pltpu.sync_copy(zer_hbm.at[sl], asy.at[sl])
            pltpu.sync_copy(zer_hbm.at[sl], asz.at[sl])

        plsc.subcore_barrier()

        rem = _NCHUNKS % _NW
        ng = jnp.where(wid < rem, _NCHUNKS // _NW + 1, _NCHUNKS // _NW)

        @pl.loop(0, ng)
        def _(g):
            cid = g * _NW + wid
            pltpu.sync_copy(a_hbm.at[pl.ds(cid * _CHUNK, _CHUNK)], aix)
            pltpu.sync_copy(b_hbm.at[pl.ds(cid * _CHUNK, _CHUNK)], bix)
            pltpu.sync_copy(f_hbm.at[pl.ds(cid * _CHUNK, _CHUNK)], fbuf)

            gathers = [
                pltpu.async_copy(psx.at[aix], pxa, sem_g),
                pltpu.async_copy(psy.at[aix], pya, sem_g),
                pltpu.async_copy(psz.at[aix], pza, sem_g),
                pltpu.async_copy(psx.at[bix], pxb, sem_g),
                pltpu.async_copy(psy.at[bix], pyb, sem_g),
                pltpu.async_copy(psz.at[bix], pzb, sem_g),
            ]
            for h in gathers:
                h.wait()

            @pl.loop(0, _CHUNK // 16)
            def _(r):
                q = pl.ds(r * 16, 16)
                vx = pxb[q] - pxa[q]
                vy = pyb[q] - pya[q]
                vz = pzb[q] - pza[q]
                d = vx * vx + vy * vy + vz * vz
                bits = lax.bitcast_convert_type(d, jnp.int32)
                y = lax.bitcast_convert_type(
                    jnp.int32(0x5F3759DF) - (bits >> 1), f32)
                y = y * (1.5 - 0.5 * d * y * y)
                y = y * (1.5 - 0.5 * d * y * y)
                y = y * (1.5 - 0.5 * d * y * y)
                sp = fbuf[q] * y
                gx = sp * vx
                gy = sp * vy
                gz = sp * vz
                fbx[q] = gx
                fby[q] = gy
                fbz[q] = gz
                fax[q] = -gx
                fay[q] = -gy
                faz[q] = -gz

            scatters = [
                pltpu.async_copy(fax, asx.at[aix], sem_s, add=True),
                pltpu.async_copy(fay, asy.at[aix], sem_s, add=True),
                pltpu.async_copy(faz, asz.at[aix], sem_s, add=True),
                pltpu.async_copy(fbx, asx.at[bix], sem_s, add=True),
                pltpu.async_copy(fby, asy.at[bix], sem_s, add=True),
                pltpu.async_copy(fbz, asz.at[bix], sem_s, add=True),
            ]
            for h in scatters:
                h.wait()

        plsc.subcore_barrier()
        base = c * 3 * _NP
        pltpu.sync_copy(asx.at[sl], out_hbm.at[pl.ds(base + r0, _NPT)])
        pltpu.sync_copy(asy.at[sl], out_hbm.at[pl.ds(base + _NP + r0, _NPT)])
        pltpu.sync_copy(asz.at[sl],
                        out_hbm.at[pl.ds(base + 2 * _NP + r0, _NPT)])

    return k(xp, yp, zp, ex, ey, ez, zeros1, a2, b2, f1)


def _tc_combine(p0, p1):
    def body(x_ref, y_ref, o_ref):
        o_ref[...] = x_ref[...] + y_ref[...]

    return pl.pallas_call(
        body,
        out_shape=jax.ShapeDtypeStruct(p0.shape, p0.dtype),
    )(p0, p1)


def kernel(points, external_forces, force, edge_index):
    pad = (0, _NP - _N)
    xp = jnp.pad(points[:, 0], pad)
    yp = jnp.pad(points[:, 1], pad)
    zp = jnp.pad(points[:, 2], pad)
    ex = jnp.pad(external_forces[:, 0], pad)
    ey = jnp.pad(external_forces[:, 1], pad)
    ez = jnp.pad(external_forces[:, 2], pad)
    zeros1 = jnp.zeros((_NP,), jnp.float32)
    a2 = edge_index[0]
    b2 = edge_index[1]
    partial = _sc_forces(xp, yp, zp, ex, ey, ez, zeros1, a2, b2, force)
    m = 3 * _NP // 128
    s = _tc_combine(partial[:3 * _NP].reshape(m, 128),
                    partial[3 * _NP:].reshape(m, 128))
    return s.reshape(3, _NP)[:, :_N].T
